# baseline (speedup 1.0000x reference)
import functools

import jax
import jax.numpy as jnp
from jax import lax
from jax.experimental import pallas as pl
from jax.experimental.pallas import tpu as pltpu
from jax.experimental.pallas import tpu_sc as plsc

_N = 10000
_E = 320000
_D = 128
_L = 16
_NC = 2
_NS = 16
_NW = _NC * _NS
_C = 80
_EPT = _E // _NW
_NCHUNK = _EPT // _C
_RPT = 624
_RTAIL = _N - _NS * _RPT
_DJ = _D // _L


def _sc_edge_kernel(g_hbm, q_hbm, aw_hbm, src_hbm, dst_hbm, out_hbm,
                    rows0_v, rows1_v, src0_v, src1_v, dst0_v, dst1_v,
                    sdst0_v, sdst1_v, aidx0_v, aidx1_v,
                    agg_sh, semg0, semg1, semqa0, semqa1, semi0, semi1,
                    semsc0, semsc1):
    i32 = jnp.int32
    cid = lax.axis_index("c")
    sid = lax.axis_index("s")
    tid = sid * i32(_NC) + cid
    iota = lax.iota(jnp.int32, _L)
    rows = [rows0_v, rows1_v]
    srcb = [src0_v, src1_v]
    dstb = [dst0_v, dst1_v]
    sdst = [sdst0_v, sdst1_v]
    aidx = [aidx0_v, aidx1_v]
    semg = [semg0, semg1]
    semqa = [semqa0, semqa1]
    semi = [semi0, semi1]
    semsc = [semsc0, semsc1]

    @plsc.parallel_loop(jnp.int32(0), jnp.int32(_C), step=jnp.int32(1),
                        unroll=4)
    def _zrow(i):
        for b in range(2):
            for j in range(_DJ):
                rows[b][i, pl.ds(j * _L, _L)] = jnp.zeros((_L,), jnp.float32)
    for b in range(2):
        for jj in range(_C // _L):
            sdst[b][pl.ds(jj * _L, _L)] = jnp.zeros((_L,), jnp.int32)

    for t in range(7):
        pltpu.sync_copy(rows0_v.at[pl.ds(0, _C)],
                        agg_sh.at[pl.ds(sid * i32(_RPT) + i32(t * _C), _C)])
    pltpu.sync_copy(rows0_v.at[pl.ds(0, _RPT - 7 * _C)],
                    agg_sh.at[pl.ds(sid * i32(_RPT) + i32(7 * _C),
                                    _RPT - 7 * _C)])

    @pl.when(sid == i32(_NS - 1))
    def _zero_tail():
        pltpu.sync_copy(rows0_v.at[pl.ds(0, _RTAIL)],
                        agg_sh.at[pl.ds(i32(_NS * _RPT), _RTAIL)])
    plsc.subcore_barrier()

    ebase = tid * i32(_EPT)

    def issue_idx(b, c):
        off = ebase + c * i32(_C)
        pltpu.async_copy(src_hbm.at[pl.ds(off, _C)], srcb[b], semi[b])
        pltpu.async_copy(dst_hbm.at[pl.ds(off, _C)], dstb[b], semi[b])

    def wait_idx(b):
        pltpu.make_async_copy(src_hbm.at[pl.ds(0, _C)], srcb[b],
                              semi[b]).wait()
        pltpu.make_async_copy(dst_hbm.at[pl.ds(0, _C)], dstb[b],
                              semi[b]).wait()

    def issue_g(b):
        pltpu.async_copy(g_hbm.at[srcb[b]], rows[b], semg[b])

    def wait_g(b):
        pltpu.make_async_copy(g_hbm.at[srcb[b]], rows[b], semg[b]).wait()

    def issue_qa(b, c):
        off = ebase + c * i32(_C)
        for jj in range(_C // _L):
            aidx[b][pl.ds(jj * _L, _L)] = off + i32(jj * _L) + iota
        pltpu.async_copy(q_hbm.at[dstb[b]], rows[b], semqa[b], add=True)
        pltpu.async_copy(aw_hbm.at[aidx[b]], rows[b], semqa[b], add=True)

    def wait_qa(b):
        pltpu.make_async_copy(q_hbm.at[dstb[b]], rows[b], semqa[b]).wait()
        pltpu.make_async_copy(aw_hbm.at[aidx[b]], rows[b], semqa[b]).wait()

    def issue_scatter(b):
        pltpu.async_copy(rows[b], agg_sh.at[sdst[b]], semsc[b], add=True)

    def wait_scatter(b):
        pltpu.make_async_copy(rows[b], agg_sh.at[sdst[b]], semsc[b]).wait()

    def compute(b):
        for jj in range(_C // _L):
            sdst[b][pl.ds(jj * _L, _L)] = dstb[b][pl.ds(jj * _L, _L)]

        @plsc.parallel_loop(jnp.int32(0), jnp.int32(_C), step=jnp.int32(1),
                            unroll=8)
        def _edge(e):
            for j in range(_DJ):
                rows[b][e, pl.ds(j * _L, _L)] = jnp.maximum(
                    rows[b][e, pl.ds(j * _L, _L)], 0.0)

    issue_scatter(0)
    issue_scatter(1)
    issue_idx(0, i32(0))
    wait_idx(0)
    wait_scatter(0)
    issue_g(0)
    issue_idx(1, i32(1))
    wait_g(0)
    issue_qa(0, i32(0))
    wait_idx(1)
    wait_scatter(1)
    issue_g(1)

    def _pair(j2, carry):
        c2 = jnp.int32(2) * j2 + i32(2)
        c3 = jnp.minimum(c2 + i32(1), i32(_NCHUNK - 1))
        wait_qa(0)
        compute(0)
        issue_scatter(0)
        issue_idx(0, c2)
        wait_g(1)
        issue_qa(1, c2 - i32(1))
        wait_idx(0)
        wait_scatter(0)
        issue_g(0)
        wait_qa(1)
        compute(1)
        issue_scatter(1)
        issue_idx(1, c3)
        wait_g(0)
        issue_qa(0, c2)
        wait_idx(1)
        wait_scatter(1)
        issue_g(1)
        return carry
    lax.fori_loop(jnp.int32(0), jnp.int32((_NCHUNK - 1) // 2), _pair, 0)

    wait_qa(0)
    compute(0)
    issue_scatter(0)
    wait_scatter(0)
    wait_g(1)

    plsc.subcore_barrier()
    for t in range(7):
        r0 = sid * i32(_RPT) + i32(t * _C)
        pltpu.sync_copy(agg_sh.at[pl.ds(r0, _C)],
                        out_hbm.at[cid, pl.ds(r0, _C)])
    r0 = sid * i32(_RPT) + i32(7 * _C)
    pltpu.sync_copy(agg_sh.at[pl.ds(r0, _RPT - 7 * _C)],
                    out_hbm.at[cid, pl.ds(r0, _RPT - 7 * _C)])

    @pl.when(sid == i32(_NS - 1))
    def _out_tail():
        pltpu.sync_copy(agg_sh.at[pl.ds(i32(_NS * _RPT), _RTAIL)],
                        out_hbm.at[cid, pl.ds(i32(_NS * _RPT), _RTAIL)])


_sc_edge = functools.partial(
    pl.kernel,
    out_type=jax.ShapeDtypeStruct((_NC, _N, _D), jnp.float32),
    mesh=plsc.VectorSubcoreMesh(core_axis_name="c", subcore_axis_name="s"),
    scratch_types=[
        pltpu.VMEM((_C, _D), jnp.float32),
        pltpu.VMEM((_C, _D), jnp.float32),
        pltpu.VMEM((_C,), jnp.int32),
        pltpu.VMEM((_C,), jnp.int32),
        pltpu.VMEM((_C,), jnp.int32),
        pltpu.VMEM((_C,), jnp.int32),
        pltpu.VMEM((_C,), jnp.int32),
        pltpu.VMEM((_C,), jnp.int32),
        pltpu.VMEM((_C,), jnp.int32),
        pltpu.VMEM((_C,), jnp.int32),
        pltpu.VMEM_SHARED((_N, _D), jnp.float32),
        pltpu.SemaphoreType.DMA,
        pltpu.SemaphoreType.DMA,
        pltpu.SemaphoreType.DMA,
        pltpu.SemaphoreType.DMA,
        pltpu.SemaphoreType.DMA,
        pltpu.SemaphoreType.DMA,
        pltpu.SemaphoreType.DMA,
        pltpu.SemaphoreType.DMA,
    ],
)(_sc_edge_kernel)


def _prep_body(x_ref, pos_ref, attr_ref, ws_ref, wp_ref, we_ref,
               g_ref, q_ref, aw_ref):
    pq = jnp.dot(pos_ref[...], wp_ref[...], preferred_element_type=jnp.float32)
    g_ref[...] = (
        jnp.dot(x_ref[...], ws_ref[...], preferred_element_type=jnp.float32)
        - pq
    )
    q_ref[...] = pq
    aw_ref[...] = jnp.dot(attr_ref[...], we_ref[...],
                          preferred_element_type=jnp.float32)


def _post_body(a0_ref, a1_ref, x_ref, wo_ref, wsf_ref, b_ref, o_ref):
    agg = a0_ref[...] + a1_ref[...]
    o_ref[...] = jnp.maximum(
        jnp.dot(agg, wo_ref[...], preferred_element_type=jnp.float32)
        + jnp.dot(x_ref[...], wsf_ref[...], preferred_element_type=jnp.float32)
        + b_ref[...],
        0.0,
    )


_BLK = 400
_BLKE = 12800


def kernel(x, pos, edge_index, edge_attr, W_src, W_edge, W_pos, W_self,
           W_out, b):
    x = x.astype(jnp.float32)
    pos = pos.astype(jnp.float32)
    src = edge_index[0].astype(jnp.int32)
    dst = edge_index[1].astype(jnp.int32)
    pos4 = jnp.pad(pos, ((0, 0), (0, 1)))
    wp4 = jnp.pad(W_pos.astype(jnp.float32), ((0, 1), (0, 0)))

    grid = _N // _BLK
    g, q, aw = pl.pallas_call(
        _prep_body,
        grid=(grid,),
        in_specs=[
            pl.BlockSpec((_BLK, _D), lambda i: (i, 0 * i)),
            pl.BlockSpec((_BLK, 4), lambda i: (i, 0 * i)),
            pl.BlockSpec((_BLKE, 4), lambda i: (i, 0 * i)),
            pl.BlockSpec((_D, _D), lambda i: (0 * i, 0 * i)),
            pl.BlockSpec((4, _D), lambda i: (0 * i, 0 * i)),
            pl.BlockSpec((4, _D), lambda i: (0 * i, 0 * i)),
        ],
        out_specs=[
            pl.BlockSpec((_BLK, _D), lambda i: (i, 0 * i)),
            pl.BlockSpec((_BLK, _D), lambda i: (i, 0 * i)),
            pl.BlockSpec((_BLKE, _D), lambda i: (i, 0 * i)),
        ],
        out_shape=[
            jax.ShapeDtypeStruct((_N, _D), jnp.float32),
            jax.ShapeDtypeStruct((_N, _D), jnp.float32),
            jax.ShapeDtypeStruct((_E, _D), jnp.float32),
        ],
    )(x, pos4, edge_attr.astype(jnp.float32), W_src.astype(jnp.float32),
      wp4, W_edge.astype(jnp.float32))

    agg2 = _sc_edge(g, q, aw, src, dst)

    out = pl.pallas_call(
        _post_body,
        grid=(grid,),
        in_specs=[
            pl.BlockSpec((_BLK, _D), lambda i: (i, 0 * i)),
            pl.BlockSpec((_BLK, _D), lambda i: (i, 0 * i)),
            pl.BlockSpec((_BLK, _D), lambda i: (i, 0 * i)),
            pl.BlockSpec((_D, _D), lambda i: (0 * i, 0 * i)),
            pl.BlockSpec((_D, _D), lambda i: (0 * i, 0 * i)),
            pl.BlockSpec((1, _D), lambda i: (0 * i, 0 * i)),
        ],
        out_specs=pl.BlockSpec((_BLK, _D), lambda i: (i, 0 * i)),
        out_shape=jax.ShapeDtypeStruct((_N, _D), jnp.float32),
    )(agg2[0], agg2[1], x, W_out.astype(jnp.float32),
      W_self.astype(jnp.float32), b.astype(jnp.float32).reshape(1, _D))
    return out


# speedup vs baseline: 1.0112x; 1.0112x over previous
import functools

import jax
import jax.numpy as jnp
from jax import lax
from jax.experimental import pallas as pl
from jax.experimental.pallas import tpu as pltpu
from jax.experimental.pallas import tpu_sc as plsc

_N = 10000
_E = 320000
_D = 128
_L = 16
_NC = 2
_NS = 16
_NW = _NC * _NS
_C = 80
_EPT = _E // _NW
_NCHUNK = _EPT // _C
_RPT = 624
_RTAIL = _N - _NS * _RPT
_DJ = _D // _L


def _sc_edge_kernel(g_hbm, q_hbm, aw_hbm, src_hbm, dst_hbm, out_hbm,
                    rows0_v, rows1_v, src0_v, src1_v, dst0_v, dst1_v,
                    sdst0_v, sdst1_v, aidx0_v, aidx1_v,
                    agg_sh, semg0, semg1, semqa0, semqa1, semi0, semi1,
                    semsc0, semsc1):
    i32 = jnp.int32
    cid = lax.axis_index("c")
    sid = lax.axis_index("s")
    tid = sid * i32(_NC) + cid
    iota = lax.iota(jnp.int32, _L)
    rows = [rows0_v, rows1_v]
    srcb = [src0_v, src1_v]
    dstb = [dst0_v, dst1_v]
    sdst = [sdst0_v, sdst1_v]
    aidx = [aidx0_v, aidx1_v]
    semg = [semg0, semg1]
    semqa = [semqa0, semqa1]
    semi = [semi0, semi1]
    semsc = [semsc0, semsc1]

    def _zrow(i, carry):
        for b in range(2):
            for j in range(_DJ):
                rows[b][i, pl.ds(j * _L, _L)] = jnp.zeros((_L,), jnp.float32)
        return carry
    lax.fori_loop(jnp.int32(0), jnp.int32(_C), _zrow, 0)
    for b in range(2):
        for jj in range(_C // _L):
            sdst[b][pl.ds(jj * _L, _L)] = jnp.zeros((_L,), jnp.int32)

    for t in range(7):
        pltpu.sync_copy(rows0_v.at[pl.ds(0, _C)],
                        agg_sh.at[pl.ds(sid * i32(_RPT) + i32(t * _C), _C)])
    pltpu.sync_copy(rows0_v.at[pl.ds(0, _RPT - 7 * _C)],
                    agg_sh.at[pl.ds(sid * i32(_RPT) + i32(7 * _C),
                                    _RPT - 7 * _C)])

    @pl.when(sid == i32(_NS - 1))
    def _zero_tail():
        pltpu.sync_copy(rows0_v.at[pl.ds(0, _RTAIL)],
                        agg_sh.at[pl.ds(i32(_NS * _RPT), _RTAIL)])
    plsc.subcore_barrier()

    ebase = tid * i32(_EPT)

    def issue_idx(b, c):
        off = ebase + c * i32(_C)
        pltpu.async_copy(src_hbm.at[pl.ds(off, _C)], srcb[b], semi[b])
        pltpu.async_copy(dst_hbm.at[pl.ds(off, _C)], dstb[b], semi[b])

    def wait_idx(b):
        pltpu.make_async_copy(src_hbm.at[pl.ds(0, _C)], srcb[b],
                              semi[b]).wait()
        pltpu.make_async_copy(dst_hbm.at[pl.ds(0, _C)], dstb[b],
                              semi[b]).wait()

    def issue_g(b):
        pltpu.async_copy(g_hbm.at[srcb[b]], rows[b], semg[b])

    def wait_g(b):
        pltpu.make_async_copy(g_hbm.at[srcb[b]], rows[b], semg[b]).wait()

    def issue_qa(b, c):
        off = ebase + c * i32(_C)
        for jj in range(_C // _L):
            aidx[b][pl.ds(jj * _L, _L)] = off + i32(jj * _L) + iota
        pltpu.async_copy(q_hbm.at[dstb[b]], rows[b], semqa[b], add=True)
        pltpu.async_copy(aw_hbm.at[aidx[b]], rows[b], semqa[b], add=True)

    def wait_qa(b):
        pltpu.make_async_copy(q_hbm.at[dstb[b]], rows[b], semqa[b]).wait()
        pltpu.make_async_copy(aw_hbm.at[aidx[b]], rows[b], semqa[b]).wait()

    def issue_scatter(b):
        pltpu.async_copy(rows[b], agg_sh.at[sdst[b]], semsc[b], add=True)

    def wait_scatter(b):
        pltpu.make_async_copy(rows[b], agg_sh.at[sdst[b]], semsc[b]).wait()

    def compute(b):
        for jj in range(_C // _L):
            sdst[b][pl.ds(jj * _L, _L)] = dstb[b][pl.ds(jj * _L, _L)]

        def _edge8(e8, carry2):
            e0 = e8 * i32(8)
            for u in range(8):
                e = e0 + i32(u)
                for j in range(_DJ):
                    rows[b][e, pl.ds(j * _L, _L)] = jnp.maximum(
                        rows[b][e, pl.ds(j * _L, _L)], 0.0)
            return carry2
        lax.fori_loop(jnp.int32(0), jnp.int32(_C // 8), _edge8, 0)

    issue_scatter(0)
    issue_scatter(1)
    issue_idx(0, i32(0))
    wait_idx(0)
    wait_scatter(0)
    issue_g(0)
    issue_idx(1, i32(1))
    wait_g(0)
    issue_qa(0, i32(0))
    wait_idx(1)
    wait_scatter(1)
    issue_g(1)

    def _pair(j2, carry):
        c2 = jnp.int32(2) * j2 + i32(2)
        c3 = jnp.minimum(c2 + i32(1), i32(_NCHUNK - 1))
        wait_qa(0)
        compute(0)
        issue_scatter(0)
        issue_idx(0, c2)
        wait_g(1)
        issue_qa(1, c2 - i32(1))
        wait_idx(0)
        wait_scatter(0)
        issue_g(0)
        wait_qa(1)
        compute(1)
        issue_scatter(1)
        issue_idx(1, c3)
        wait_g(0)
        issue_qa(0, c2)
        wait_idx(1)
        wait_scatter(1)
        issue_g(1)
        return carry
    lax.fori_loop(jnp.int32(0), jnp.int32((_NCHUNK - 1) // 2), _pair, 0)

    wait_qa(0)
    compute(0)
    issue_scatter(0)
    wait_scatter(0)
    wait_g(1)

    plsc.subcore_barrier()
    for t in range(7):
        r0 = sid * i32(_RPT) + i32(t * _C)
        pltpu.sync_copy(agg_sh.at[pl.ds(r0, _C)],
                        out_hbm.at[cid, pl.ds(r0, _C)])
    r0 = sid * i32(_RPT) + i32(7 * _C)
    pltpu.sync_copy(agg_sh.at[pl.ds(r0, _RPT - 7 * _C)],
                    out_hbm.at[cid, pl.ds(r0, _RPT - 7 * _C)])

    @pl.when(sid == i32(_NS - 1))
    def _out_tail():
        pltpu.sync_copy(agg_sh.at[pl.ds(i32(_NS * _RPT), _RTAIL)],
                        out_hbm.at[cid, pl.ds(i32(_NS * _RPT), _RTAIL)])


_sc_edge = functools.partial(
    pl.kernel,
    out_type=jax.ShapeDtypeStruct((_NC, _N, _D), jnp.float32),
    mesh=plsc.VectorSubcoreMesh(core_axis_name="c", subcore_axis_name="s"),
    scratch_types=[
        pltpu.VMEM((_C, _D), jnp.float32),
        pltpu.VMEM((_C, _D), jnp.float32),
        pltpu.VMEM((_C,), jnp.int32),
        pltpu.VMEM((_C,), jnp.int32),
        pltpu.VMEM((_C,), jnp.int32),
        pltpu.VMEM((_C,), jnp.int32),
        pltpu.VMEM((_C,), jnp.int32),
        pltpu.VMEM((_C,), jnp.int32),
        pltpu.VMEM((_C,), jnp.int32),
        pltpu.VMEM((_C,), jnp.int32),
        pltpu.VMEM_SHARED((_N, _D), jnp.float32),
        pltpu.SemaphoreType.DMA,
        pltpu.SemaphoreType.DMA,
        pltpu.SemaphoreType.DMA,
        pltpu.SemaphoreType.DMA,
        pltpu.SemaphoreType.DMA,
        pltpu.SemaphoreType.DMA,
        pltpu.SemaphoreType.DMA,
        pltpu.SemaphoreType.DMA,
    ],
)(_sc_edge_kernel)


def _prep_body(x_ref, pos_ref, ws_ref, wp_ref, g_ref, q_ref):
    pq = jnp.dot(pos_ref[...], wp_ref[...], preferred_element_type=jnp.float32)
    g_ref[...] = (
        jnp.dot(x_ref[...], ws_ref[...], preferred_element_type=jnp.float32)
        - pq
    )
    q_ref[...] = pq


def _aw_body(attr_ref, we_ref, aw_ref):
    aw_ref[...] = jnp.dot(attr_ref[...], we_ref[...],
                          preferred_element_type=jnp.float32)


def _post_body(a0_ref, a1_ref, x_ref, wo_ref, wsf_ref, b_ref, o_ref):
    agg = a0_ref[...] + a1_ref[...]
    o_ref[...] = jnp.maximum(
        jnp.dot(agg, wo_ref[...], preferred_element_type=jnp.float32)
        + jnp.dot(x_ref[...], wsf_ref[...], preferred_element_type=jnp.float32)
        + b_ref[...],
        0.0,
    )


_BLK = 2000
_BLKE = 16000


def kernel(x, pos, edge_index, edge_attr, W_src, W_edge, W_pos, W_self,
           W_out, b):
    x = x.astype(jnp.float32)
    pos = pos.astype(jnp.float32)
    src = edge_index[0].astype(jnp.int32)
    dst = edge_index[1].astype(jnp.int32)
    pos4 = jnp.pad(pos, ((0, 0), (0, 1)))
    wp4 = jnp.pad(W_pos.astype(jnp.float32), ((0, 1), (0, 0)))

    grid = _N // _BLK
    g, q = pl.pallas_call(
        _prep_body,
        grid=(grid,),
        in_specs=[
            pl.BlockSpec((_BLK, _D), lambda i: (i, 0 * i)),
            pl.BlockSpec((_BLK, 4), lambda i: (i, 0 * i)),
            pl.BlockSpec((_D, _D), lambda i: (0 * i, 0 * i)),
            pl.BlockSpec((4, _D), lambda i: (0 * i, 0 * i)),
        ],
        out_specs=[
            pl.BlockSpec((_BLK, _D), lambda i: (i, 0 * i)),
            pl.BlockSpec((_BLK, _D), lambda i: (i, 0 * i)),
        ],
        out_shape=[
            jax.ShapeDtypeStruct((_N, _D), jnp.float32),
            jax.ShapeDtypeStruct((_N, _D), jnp.float32),
        ],
    )(x, pos4, W_src.astype(jnp.float32), wp4)

    aw = pl.pallas_call(
        _aw_body,
        grid=(_E // _BLKE,),
        in_specs=[
            pl.BlockSpec((_BLKE, 4), lambda i: (i, 0 * i)),
            pl.BlockSpec((4, _D), lambda i: (0 * i, 0 * i)),
        ],
        out_specs=pl.BlockSpec((_BLKE, _D), lambda i: (i, 0 * i)),
        out_shape=jax.ShapeDtypeStruct((_E, _D), jnp.float32),
    )(edge_attr.astype(jnp.float32), W_edge.astype(jnp.float32))

    agg2 = _sc_edge(g, q, aw, src, dst)

    out = pl.pallas_call(
        _post_body,
        grid=(grid,),
        in_specs=[
            pl.BlockSpec((_BLK, _D), lambda i: (i, 0 * i)),
            pl.BlockSpec((_BLK, _D), lambda i: (i, 0 * i)),
            pl.BlockSpec((_BLK, _D), lambda i: (i, 0 * i)),
            pl.BlockSpec((_D, _D), lambda i: (0 * i, 0 * i)),
            pl.BlockSpec((_D, _D), lambda i: (0 * i, 0 * i)),
            pl.BlockSpec((1, _D), lambda i: (0 * i, 0 * i)),
        ],
        out_specs=pl.BlockSpec((_BLK, _D), lambda i: (i, 0 * i)),
        out_shape=jax.ShapeDtypeStruct((_N, _D), jnp.float32),
    )(agg2[0], agg2[1], x, W_out.astype(jnp.float32),
      W_self.astype(jnp.float32), b.astype(jnp.float32).reshape(1, _D))
    return out
